# Initial kernel scaffold; baseline (speedup 1.0000x reference)
import jax
import jax.numpy as jnp
from jax.experimental import pallas as pl


def kernel(x, input_images):
    raise NotImplementedError("write your pallas kernel here")


# speedup vs baseline: 1.9123x; 1.9123x over previous
import jax
import jax.numpy as jnp
from jax.experimental import pallas as pl
from jax.experimental.pallas import tpu as pltpu

IMG = 384
PATCH = 16
FEAT = IMG // PATCH
NUM_PATCHES = FEAT * FEAT
CUTOFF = NUM_PATCHES // 2

_I32_MIN = -(2 ** 31)
_I32_MAXP = (1 << 31) - 1


def _body(att_ref, img_ref, out_ref, mask_ref):
    b = pl.program_id(0)

    @pl.when(b == 0)
    def _prologue():
        a = jnp.sum(att_ref[...], axis=1)
        bits = jax.lax.bitcast_convert_type(a, jnp.int32)
        skey = jnp.where(bits >= 0, bits, bits ^ jnp.int32(_I32_MAXP))
        tu = jnp.zeros((a.shape[0], 1, 1), jnp.int32)
        for bit in range(31, -1, -1):
            bp = jnp.int32(_I32_MIN) if bit == 31 else jnp.int32(1 << bit)
            cand_u = tu | bp
            cand_s = cand_u ^ jnp.int32(_I32_MIN)
            cnt = jnp.sum((skey >= cand_s).astype(jnp.int32),
                          axis=(1, 2), keepdims=True)
            tu = jnp.where(cnt >= CUTOFF, cand_u, tu)
        ts = tu ^ jnp.int32(_I32_MIN)
        thr_bits = jnp.where(ts >= 0, ts, ts ^ jnp.int32(_I32_MAXP))
        thr = jax.lax.bitcast_convert_type(thr_bits, jnp.float32)
        mask_ref[...] = jnp.where(a > thr, 1.0, 0.0).astype(jnp.float32)

    m = mask_ref[b]
    p_cols = jnp.where(
        jax.lax.broadcasted_iota(jnp.int32, (FEAT, IMG), 1) // PATCH
        == jax.lax.broadcasted_iota(jnp.int32, (FEAT, IMG), 0),
        1.0, 0.0).astype(jnp.float32)
    p_rows = jnp.where(
        jax.lax.broadcasted_iota(jnp.int32, (IMG, FEAT), 0) // PATCH
        == jax.lax.broadcasted_iota(jnp.int32, (IMG, FEAT), 1),
        1.0, 0.0).astype(jnp.float32)
    mp = jax.lax.dot_general(m, p_cols, (((1,), (0,)), ((), ())),
                             preferred_element_type=jnp.float32)
    m_full = jax.lax.dot_general(p_rows, mp, (((1,), (0,)), ((), ())),
                                 preferred_element_type=jnp.float32)
    out_ref[0] = img_ref[0] * m_full[None, :, :]


def kernel(x, input_images):
    B, NH = x.shape[0], x.shape[1]
    att = x[:, :, 0, 1:].reshape(B, NH, FEAT, FEAT)
    return pl.pallas_call(
        _body,
        grid=(B,),
        in_specs=[
            pl.BlockSpec((B, NH, FEAT, FEAT), lambda b: (0, 0, 0, 0)),
            pl.BlockSpec((1, 3, IMG, IMG), lambda b: (b, 0, 0, 0)),
        ],
        out_specs=pl.BlockSpec((1, 3, IMG, IMG), lambda b: (b, 0, 0, 0)),
        out_shape=jax.ShapeDtypeStruct(input_images.shape, input_images.dtype),
        scratch_shapes=[pltpu.VMEM((B, FEAT, FEAT), jnp.float32)],
    )(att, input_images)


# speedup vs baseline: 2.1240x; 1.1107x over previous
import jax
import jax.numpy as jnp
from jax.experimental import pallas as pl
from jax.experimental.pallas import tpu as pltpu

IMG = 384
PATCH = 16
FEAT = IMG // PATCH
NUM_PATCHES = FEAT * FEAT
CUTOFF = NUM_PATCHES // 2

_I32_MIN = -(2 ** 31)
_I32_MAXP = (1 << 31) - 1


def _body(att_ref, img_ref, out_ref, mask_ref):
    b = pl.program_id(0)

    @pl.when(b == 0)
    def _prologue():
        a = jnp.sum(att_ref[...], axis=1)
        bits = jax.lax.bitcast_convert_type(a, jnp.int32)
        skey = jnp.where(bits >= 0, bits, bits ^ jnp.int32(_I32_MAXP))
        tu = jnp.zeros((a.shape[0], 1, 1), jnp.int32)
        for bit in range(31, -1, -1):
            bp = jnp.int32(_I32_MIN) if bit == 31 else jnp.int32(1 << bit)
            cand_u = tu | bp
            cand_s = cand_u ^ jnp.int32(_I32_MIN)
            cnt = jnp.sum((skey >= cand_s).astype(jnp.int32),
                          axis=(1, 2), keepdims=True)
            tu = jnp.where(cnt >= CUTOFF, cand_u, tu)
        ts = tu ^ jnp.int32(_I32_MIN)
        thr_bits = jnp.where(ts >= 0, ts, ts ^ jnp.int32(_I32_MAXP))
        thr = jax.lax.bitcast_convert_type(thr_bits, jnp.float32)
        mask_ref[...] = jnp.where(a > thr, 1.0, 0.0).astype(jnp.float32)

    m = mask_ref[2 * b]
    m2 = mask_ref[2 * b + 1]
    p_cols = jnp.where(
        jax.lax.broadcasted_iota(jnp.int32, (FEAT, IMG), 1) // PATCH
        == jax.lax.broadcasted_iota(jnp.int32, (FEAT, IMG), 0),
        1.0, 0.0).astype(jnp.float32)
    p_rows = jnp.where(
        jax.lax.broadcasted_iota(jnp.int32, (IMG, FEAT), 0) // PATCH
        == jax.lax.broadcasted_iota(jnp.int32, (IMG, FEAT), 1),
        1.0, 0.0).astype(jnp.float32)
    mp = jax.lax.dot_general(m, p_cols, (((1,), (0,)), ((), ())),
                             preferred_element_type=jnp.float32)
    m_full = jax.lax.dot_general(p_rows, mp, (((1,), (0,)), ((), ())),
                                 preferred_element_type=jnp.float32)
    mp2 = jax.lax.dot_general(m2, p_cols, (((1,), (0,)), ((), ())),
                              preferred_element_type=jnp.float32)
    m_full2 = jax.lax.dot_general(p_rows, mp2, (((1,), (0,)), ((), ())),
                                  preferred_element_type=jnp.float32)
    out_ref[0] = img_ref[0] * m_full[None, :, :]
    out_ref[1] = img_ref[1] * m_full2[None, :, :]


def kernel(x, input_images):
    B, NH = x.shape[0], x.shape[1]
    att = x[:, :, 0, 1:].reshape(B, NH, FEAT, FEAT)
    return pl.pallas_call(
        _body,
        grid=(B // 2,),
        in_specs=[
            pl.BlockSpec((B, NH, FEAT, FEAT), lambda b: (0, 0, 0, 0)),
            pl.BlockSpec((2, 3, IMG, IMG), lambda b: (b, 0, 0, 0)),
        ],
        out_specs=pl.BlockSpec((2, 3, IMG, IMG), lambda b: (b, 0, 0, 0)),
        out_shape=jax.ShapeDtypeStruct(input_images.shape, input_images.dtype),
        scratch_shapes=[pltpu.VMEM((B, FEAT, FEAT), jnp.float32)],
    )(att, input_images)


# speedup vs baseline: 2.2239x; 1.0470x over previous
import jax
import jax.numpy as jnp
from jax.experimental import pallas as pl
from jax.experimental.pallas import tpu as pltpu

IMG = 384
PATCH = 16
FEAT = IMG // PATCH
NUM_PATCHES = FEAT * FEAT
CUTOFF = NUM_PATCHES // 2
BB = 4

_I32_MIN = -(2 ** 31)
_I32_MAXP = (1 << 31) - 1


def _body(att_ref, img_ref, out_ref, mask_ref):
    b = pl.program_id(0)

    @pl.when(b == 0)
    def _prologue():
        a = jnp.sum(att_ref[...], axis=1)
        bits = jax.lax.bitcast_convert_type(a, jnp.int32)
        skey = jnp.where(bits >= 0, bits, bits ^ jnp.int32(_I32_MAXP))
        tu = jnp.zeros((a.shape[0], 1, 1), jnp.int32)
        for bit in range(31, -1, -1):
            bp = jnp.int32(_I32_MIN) if bit == 31 else jnp.int32(1 << bit)
            cand_u = tu | bp
            cand_s = cand_u ^ jnp.int32(_I32_MIN)
            cnt = jnp.sum((skey >= cand_s).astype(jnp.int32),
                          axis=(1, 2), keepdims=True)
            tu = jnp.where(cnt >= CUTOFF, cand_u, tu)
        ts = tu ^ jnp.int32(_I32_MIN)
        thr_bits = jnp.where(ts >= 0, ts, ts ^ jnp.int32(_I32_MAXP))
        thr = jax.lax.bitcast_convert_type(thr_bits, jnp.float32)
        mask_ref[...] = jnp.where(a > thr, 1.0, 0.0).astype(jnp.float32)

    pass
    p_cols = jnp.where(
        jax.lax.broadcasted_iota(jnp.int32, (FEAT, IMG), 1) // PATCH
        == jax.lax.broadcasted_iota(jnp.int32, (FEAT, IMG), 0),
        1.0, 0.0).astype(jnp.float32)
    p_rows = jnp.where(
        jax.lax.broadcasted_iota(jnp.int32, (IMG, FEAT), 0) // PATCH
        == jax.lax.broadcasted_iota(jnp.int32, (IMG, FEAT), 1),
        1.0, 0.0).astype(jnp.float32)
    for j in range(BB):
        m = mask_ref[BB * b + j]
        mp = jax.lax.dot_general(m, p_cols, (((1,), (0,)), ((), ())),
                                 preferred_element_type=jnp.float32)
        m_full = jax.lax.dot_general(p_rows, mp, (((1,), (0,)), ((), ())),
                                     preferred_element_type=jnp.float32)
        out_ref[j] = img_ref[j] * m_full[None, :, :]


def kernel(x, input_images):
    B, NH = x.shape[0], x.shape[1]
    att = x[:, :, 0, 1:].reshape(B, NH, FEAT, FEAT)
    return pl.pallas_call(
        _body,
        grid=(B // BB,),
        in_specs=[
            pl.BlockSpec((B, NH, FEAT, FEAT), lambda b: (0, 0, 0, 0)),
            pl.BlockSpec((BB, 3, IMG, IMG), lambda b: (b, 0, 0, 0)),
        ],
        out_specs=pl.BlockSpec((BB, 3, IMG, IMG), lambda b: (b, 0, 0, 0)),
        out_shape=jax.ShapeDtypeStruct(input_images.shape, input_images.dtype),
        scratch_shapes=[pltpu.VMEM((B, FEAT, FEAT), jnp.float32)],
    )(att, input_images)
